# Initial kernel scaffold; baseline (speedup 1.0000x reference)
import jax
import jax.numpy as jnp
from jax.experimental import pallas as pl


def kernel(input, emb, W3, b3, W5, b5, W7, b7, label_W, label_b):
    raise NotImplementedError("write your pallas kernel here")


# speedup vs baseline: 10.2456x; 10.2456x over previous
import functools

import jax
import jax.numpy as jnp
from jax import lax
from jax.experimental import pallas as pl
from jax.experimental.pallas import tpu as pltpu
from jax.experimental.pallas import tpu_sc as plsc

VOCAB = 100000
EMB = 300
EMBP = 304
HID = 512
OUT = 100
B = 64
S = 512
PAD = 8
TOK = B * S

NC = 2
NS = 16
NW = NC * NS
ROWS_PER_W = TOK // NW
G_CHUNK = 128


def _sc_gather_body(table_hbm, idx_hbm, out_hbm, idx_v, rows_v, sem):
    wid = lax.axis_index("s") * NC + lax.axis_index("c")
    base = wid * ROWS_PER_W

    def chunk(i, carry):
        off = base + i * G_CHUNK
        pltpu.sync_copy(idx_hbm.at[pl.ds(off, G_CHUNK)], idx_v)
        pltpu.async_copy(table_hbm.at[idx_v], rows_v, sem).wait()
        pltpu.sync_copy(rows_v, out_hbm.at[pl.ds(off, G_CHUNK)])
        return carry

    lax.fori_loop(0, ROWS_PER_W // G_CHUNK, chunk, 0)


def _sc_gather(table, idx_flat):
    f = pl.kernel(
        _sc_gather_body,
        out_type=jax.ShapeDtypeStruct((TOK, EMBP), jnp.float32),
        mesh=plsc.VectorSubcoreMesh(core_axis_name="c", subcore_axis_name="s"),
        scratch_types=[
            pltpu.VMEM((G_CHUNK,), jnp.int32),
            pltpu.VMEM((G_CHUNK, EMBP), jnp.float32),
            pltpu.SemaphoreType.DMA,
        ],
        compiler_params=pltpu.CompilerParams(use_tc_tiling_on_sc=False),
    )
    return f(table, idx_flat)


RB = 2


def _conv_body(x_ref, wa_ref, wb_ref, wc_ref, bias_ref, out_ref):
    M = RB * S
    x = x_ref[...].reshape(M, EMBP)
    xp = jnp.concatenate(
        [x, jnp.zeros((PAD, EMBP), dtype=x.dtype)], axis=0
    )

    ya = jnp.zeros((M, 3 * HID), dtype=jnp.float32)
    for j in range(3):
        ya = ya + jnp.dot(xp[j:j + M, :], wa_ref[j],
                          preferred_element_type=jnp.float32)
    yb = jnp.zeros((M, 2 * HID), dtype=jnp.float32)
    for j in range(3, 5):
        yb = yb + jnp.dot(xp[j:j + M, :], wb_ref[j - 3],
                          preferred_element_type=jnp.float32)
    yc = jnp.zeros((M, HID), dtype=jnp.float32)
    for j in range(5, 7):
        yc = yc + jnp.dot(xp[j:j + M, :], wc_ref[j - 5],
                          preferred_element_type=jnp.float32)

    bias = bias_ref[0]
    y3 = ya[:, :HID] + bias[:HID]
    y5 = ya[:, HID:2 * HID] + yb[:, :HID] + bias[HID:2 * HID]
    y7 = ya[:, 2 * HID:] + yb[:, HID:] + yc + bias[2 * HID:]

    t = lax.broadcasted_iota(jnp.int32, (M, 1), 0) % S
    m3 = jnp.max(jnp.where(t <= S - 3, jax.nn.relu(y3), 0.0)
                 .reshape(RB, S, HID), axis=1)
    m5 = jnp.max(jnp.where(t <= S - 5, jax.nn.relu(y5), 0.0)
                 .reshape(RB, S, HID), axis=1)
    m7 = jnp.max(jnp.where(t <= S - 7, jax.nn.relu(y7), 0.0)
                 .reshape(RB, S, HID), axis=1)
    out_ref[:, 0, :] = jnp.concatenate([m3, m5, m7], axis=1)


def _conv_doc(xg, wa, wb, wc, bias):
    return pl.pallas_call(
        _conv_body,
        grid=(B // RB,),
        in_specs=[
            pl.BlockSpec((RB, S, EMBP), lambda i: (i, 0, 0)),
            pl.BlockSpec((3, EMBP, 3 * HID), lambda i: (0, 0, 0)),
            pl.BlockSpec((2, EMBP, 2 * HID), lambda i: (0, 0, 0)),
            pl.BlockSpec((2, EMBP, HID), lambda i: (0, 0, 0)),
            pl.BlockSpec((1, 3 * HID), lambda i: (0, 0)),
        ],
        out_specs=pl.BlockSpec((RB, 1, 3 * HID), lambda i: (i, 0, 0)),
        out_shape=jax.ShapeDtypeStruct((B, 1, 3 * HID), jnp.float32),
    )(xg, wa, wb, wc, bias).reshape(B, 3 * HID)


def _linear_body(doc_ref, w_ref, b_ref, out_ref):
    out_ref[...] = (
        jnp.dot(doc_ref[...], w_ref[...], preferred_element_type=jnp.float32)
        + b_ref[0]
    )


def _linear(doc, wt, bias):
    return pl.pallas_call(
        _linear_body,
        out_shape=jax.ShapeDtypeStruct((B, OUT), jnp.float32),
    )(doc, wt, bias)


def kernel(input, emb, W3, b3, W5, b5, W7, b7, label_W, label_b):
    idx_flat = input.reshape(TOK)

    p3 = jnp.transpose(W3[:, 0, :, :], (1, 2, 0))
    p5 = jnp.transpose(W5[:, 0, :, :], (1, 2, 0))
    p7 = jnp.transpose(W7[:, 0, :, :], (1, 2, 0))
    wa = jnp.concatenate([p3[:3], p5[:3], p7[:3]], axis=2)
    wb = jnp.concatenate([p5[3:5], p7[3:5]], axis=2)
    wc = p7[5:7]
    zpad = ((0, 0), (0, EMBP - EMB), (0, 0))
    wa, wb, wc = jnp.pad(wa, zpad), jnp.pad(wb, zpad), jnp.pad(wc, zpad)
    bias = jnp.concatenate([b3, b5, b7])[None, :]

    embp = jnp.pad(emb, ((0, 0), (0, EMBP - EMB)))
    xg = _sc_gather(embp, idx_flat).reshape(B, S, EMBP)
    doc = _conv_doc(
        xg.astype(jnp.bfloat16),
        wa.astype(jnp.bfloat16),
        wb.astype(jnp.bfloat16),
        wc.astype(jnp.bfloat16),
        bias,
    )
    return _linear(doc, label_W.T, label_b[None, :])
